# baseline (speedup 1.0000x reference)
import functools

import jax
import jax.numpy as jnp
from jax import lax
from jax.experimental import pallas as pl
from jax.experimental.pallas import tpu as pltpu
from jax.experimental.pallas import tpu_sc as plsc

N_SEG = 1_000_000
U_DIM = 8
N_QUERIES = 1_048_576
ROW = 16
M = N_SEG // ROW
U2 = N_SEG * U_DIM // 16
NC, NS, L = 2, 16, 16
NW = NC * NS
QW = N_QUERIES // NW
B = 1024
NB = QW // B
CH = 128
NCH = B // CH

_COARSE_STEPS = 16
_LUT_STEPS = 7
_FINE_STEPS = 4


VPC = CH // L
ILP = 4
NBUCKET = 2048
BF = NBUCKET / 1000.0
NPV = (NBUCKET + L) // L


def _body(t_hbm, c_hbm, ts2_hbm, us8_hbm, out_hbm,
          c_v, tq, ridx, rows, gidx, cbuf, urows, obuf, p_v,
          sem, semg, semt, semo):
    wid = lax.axis_index("s") * NC + lax.axis_index("c")
    pltpu.sync_copy(c_hbm, c_v)
    lanes = lax.iota(jnp.int32, L)
    qbase = wid * QW
    OB = B * U_DIM

    def t_copy(b, slot):
        return pltpu.make_async_copy(
            t_hbm.at[pl.ds(qbase + b * B, B)],
            tq.at[pl.ds(slot * B, B)], semt.at[slot])

    def o_copy(b, slot):
        return pltpu.make_async_copy(
            obuf.at[pl.ds(slot * OB, OB)],
            out_hbm.at[pl.ds((qbase + b * B) * U_DIM, OB)], semo.at[slot])

    def p_vec(v, _):
        qv = v * L + lanes
        lo = jnp.zeros((L,), jnp.int32)
        hi = jnp.full((L,), M, jnp.int32)
        for _s in range(_COARSE_STEPS):
            mid = jnp.minimum((lo + hi) >> 1, M - 1)
            bm = (plsc.load_gather(c_v, [mid]) * BF).astype(jnp.int32)
            le = bm < qv
            lo = jnp.where(le, mid + 1, lo)
            hi = jnp.where(le, hi, mid)
        p_v[pl.ds(v * L, L)] = lo
        return 0

    lax.fori_loop(0, NPV, p_vec, 0)

    t_copy(0, 0).start()

    def batch_body(b, _):
        tslot = b & 1
        qoff = qbase + b * B

        @pl.when(b + 1 < NB)
        def _():
            t_copy(b + 1, 1 - tslot).start()

        t_copy(b, tslot).wait()

        @pl.when(b >= 2)
        def _():
            o_copy(b - 2, tslot).wait()

        tq0 = tslot * B
        ob0 = tslot * OB

        def coarse_chunk(ch, _):
            for g in range(VPC // ILP):
                v0 = ch * VPC + g * ILP
                tv = [tq[pl.ds(tq0 + (v0 + i) * L, L)] for i in range(ILP)]
                qb = [(tv[i] * BF).astype(jnp.int32) for i in range(ILP)]
                lo0 = tuple(plsc.load_gather(p_v, [qb[i]]) for i in range(ILP))
                hi0 = tuple(plsc.load_gather(p_v, [qb[i] + 1]) for i in range(ILP))

                lo, hi = list(lo0), list(hi0)
                for _s in range(_LUT_STEPS):
                    for i in range(ILP):
                        mid = jnp.minimum((lo[i] + hi[i]) >> 1, M - 1)
                        le = plsc.load_gather(c_v, [mid]) <= tv[i]
                        lo[i] = jnp.where(le, mid + 1, lo[i])
                        hi[i] = jnp.where(le, hi[i], mid)

                def w_cond(carry):
                    lo, hi = carry
                    d = hi[0] - lo[0]
                    for i in range(1, ILP):
                        d = jnp.maximum(d, hi[i] - lo[i])
                    return jnp.max(d) > 0

                def w_body(carry):
                    lo, hi = carry
                    lo, hi = list(lo), list(hi)
                    for i in range(ILP):
                        mid = jnp.minimum((lo[i] + hi[i]) >> 1, M - 1)
                        le = plsc.load_gather(c_v, [mid]) <= tv[i]
                        lo[i] = jnp.where(le, mid + 1, lo[i])
                        hi[i] = jnp.where(le, hi[i], mid)
                    return tuple(lo), tuple(hi)

                lo, hi = lax.while_loop(w_cond, w_body,
                                        (tuple(lo), tuple(hi)))
                for i in range(ILP):
                    ridx[pl.ds((v0 + i) * L, L)] = jnp.maximum(lo[i] - 1, 0)
            pltpu.async_copy(ts2_hbm.at[ridx.at[pl.ds(ch * CH, CH)]],
                             rows.at[pl.ds(ch * CH, CH)], sem.at[ch])
            return 0

        lax.fori_loop(0, NCH, coarse_chunk, 0)

        def fine_chunk(ch, _):
            pltpu.make_async_copy(ts2_hbm.at[ridx.at[pl.ds(ch * CH, CH)]],
                                  rows.at[pl.ds(ch * CH, CH)],
                                  sem.at[ch]).wait()
            for g in range(VPC // ILP):
                v0 = ch * VPC + g * ILP
                tv = [tq[pl.ds(tq0 + (v0 + i) * L, L)] for i in range(ILP)]
                qv = [(v0 + i) * L + lanes for i in range(ILP)]
                lo = [jnp.ones((L,), jnp.int32)] * ILP
                hi = [jnp.full((L,), ROW, jnp.int32)] * ILP
                for _s in range(_FINE_STEPS):
                    for i in range(ILP):
                        mid = jnp.minimum((lo[i] + hi[i]) >> 1, ROW - 1)
                        le = plsc.load_gather(rows, [qv[i], mid]) <= tv[i]
                        lo[i] = jnp.where(le, mid + 1, lo[i])
                        hi[i] = jnp.where(le, hi[i], mid)
                for i in range(ILP):
                    rc = ridx[pl.ds((v0 + i) * L, L)]
                    fi = jnp.clip(rc * ROW + lo[i] - 1, 0, N_SEG - 1)
                    gidx[pl.ds((v0 + i) * L, L)] = fi
            pltpu.async_copy(us8_hbm.at[gidx.at[pl.ds(ch * CH, CH)]],
                             urows.at[pl.ds(ch * CH, CH)], semg.at[ch])
            return 0

        lax.fori_loop(0, NCH, fine_chunk, 0)

        jl = lanes & 7
        hl = lanes >> 3

        def out_chunk(ch, _):
            pltpu.make_async_copy(us8_hbm.at[gidx.at[pl.ds(ch * CH, CH)]],
                                  urows.at[pl.ds(ch * CH, CH)],
                                  semg.at[ch]).wait()

            def pair_vec(o, _):
                for u in range(4):
                    pp = o * 4 + u
                    q0 = ch * CH + 2 * pp
                    vals = plsc.load_gather(urows, [q0 + hl, jl])
                    plsc.store_scatter(cbuf, [9 * (2 * pp + hl) + jl], vals)
                return 0

            lax.fori_loop(0, CH // 8, pair_vec, 0)

            def bvec(o, _):
                for j in range(U_DIM):
                    val = plsc.load_gather(cbuf, [9 * (o * L + lanes) + j])
                    obuf[pl.ds(ob0 + ch * CH * U_DIM + j * 128 + o * L, L)] = val
                return 0

            lax.fori_loop(0, CH // L, bvec, 0)
            return 0

        lax.fori_loop(0, NCH, out_chunk, 0)
        o_copy(b, tslot).start()
        return 0

    lax.fori_loop(0, NB, batch_body, 0)
    o_copy(NB - 2, (NB - 2) & 1).wait()
    o_copy(NB - 1, (NB - 1) & 1).wait()


NBLK = N_SEG // 128
BASE_BLK = NBLK // NW
EXTRA = NBLK - BASE_BLK * NW
TAIL_ROWS = N_SEG - NBLK * 128
TAIL_F = TAIL_ROWS * U_DIM
TAIL_OFF = NBLK * 128 * U_DIM


BPC = 32
FULL_CHUNKS = BASE_BLK // BPC
REM_BLK = BASE_BLK - FULL_CHUNKS * BPC


def _relayout_body(usT_hbm, tail_hbm, out_hbm, ub, ob, tb, sem):
    wid = lax.axis_index("s") * NC + lax.axis_index("c")
    start = wid * BASE_BLK
    lanes = lax.iota(jnp.int32, L)
    jl = lanes & 7
    col0 = lanes >> 3

    def chunk(k0, nblk):
        pltpu.sync_copy(usT_hbm.at[:, pl.ds(k0 * 128, nblk * 128)],
                        ub.at[:, pl.ds(0, nblk * 128)])

        def vec_body(o, _):
            for u in range(8):
                v = o * 8 + u
                kk = v >> 6
                ob[pl.ds(v * L, L)] = plsc.load_gather(
                    ub, [jl, col0 + 2 * (v & 63) + (kk << 7)])
            return 0

        lax.fori_loop(0, nblk * 8, vec_body, 0)
        pltpu.sync_copy(ob.at[pl.ds(0, nblk * 1024)],
                        out_hbm.at[pl.ds(k0 * 1024, nblk * 1024)])

    def full_body(i, _):
        chunk(start + i * BPC, BPC)
        return 0

    lax.fori_loop(0, FULL_CHUNKS, full_body, 0)
    chunk(start + FULL_CHUNKS * BPC, REM_BLK)

    @pl.when(wid >= NW - EXTRA)
    def _():
        chunk(NW * BASE_BLK + (wid - (NW - EXTRA)), 1)

    @pl.when(wid == NW - 1)
    def _():
        pltpu.sync_copy(tail_hbm, tb)
        pltpu.sync_copy(tb, out_hbm.at[pl.ds(TAIL_OFF, TAIL_F)])


@jax.jit
def _relayout(usT, tail):
    mesh = plsc.VectorSubcoreMesh(core_axis_name="c", subcore_axis_name="s")
    return pl.kernel(
        _relayout_body,
        out_type=jax.ShapeDtypeStruct((N_SEG * U_DIM,), jnp.float32),
        mesh=mesh,
        compiler_params=pltpu.CompilerParams(
            needs_layout_passes=False, use_tc_tiling_on_sc=True),
        scratch_types=[
            pltpu.VMEM((U_DIM, BPC * 128 + 2), jnp.float32),
            pltpu.VMEM((BPC * 1024,), jnp.float32),
            pltpu.VMEM((TAIL_F,), jnp.float32),
            pltpu.SemaphoreType.DMA,
        ],
    )(usT, tail)


@jax.jit
def _run(t, c, ts2, us8):
    mesh = plsc.VectorSubcoreMesh(core_axis_name="c", subcore_axis_name="s")
    return pl.kernel(
        _body,
        out_type=jax.ShapeDtypeStruct((N_QUERIES * U_DIM,), jnp.float32),
        mesh=mesh,
        compiler_params=pltpu.CompilerParams(
            needs_layout_passes=False, use_tc_tiling_on_sc=False),
        scratch_types=[
            pltpu.VMEM((M,), jnp.float32),
            pltpu.VMEM((2 * B,), jnp.float32),
            pltpu.VMEM((B,), jnp.int32),
            pltpu.VMEM((B, ROW), jnp.float32),
            pltpu.VMEM((B,), jnp.int32),
            pltpu.VMEM((9 * CH,), jnp.float32),
            pltpu.VMEM((B, U_DIM), jnp.float32),
            pltpu.VMEM((2 * B * U_DIM,), jnp.float32),
            pltpu.VMEM((NPV * L,), jnp.int32),
            pltpu.SemaphoreType.DMA((NCH,)),
            pltpu.SemaphoreType.DMA((NCH,)),
            pltpu.SemaphoreType.DMA((2,)),
            pltpu.SemaphoreType.DMA((2,)),
        ],
    )(t, c, ts2, us8)


def kernel(t, x, ts, us):
    c = ts[::ROW]
    ts2 = ts.reshape(M, ROW)
    usT = us.T
    tail = us[NBLK * 128:].reshape(TAIL_F)
    us8 = _relayout(usT, tail).reshape(N_SEG, U_DIM)
    o = _run(t, c, ts2, us8)
    o3 = o.reshape(N_QUERIES // 128, U_DIM, 128)
    return o3.transpose(0, 2, 1).reshape(N_QUERIES, U_DIM)


# speedup vs baseline: 1.0524x; 1.0524x over previous
import functools

import jax
import jax.numpy as jnp
from jax import lax
from jax.experimental import pallas as pl
from jax.experimental.pallas import tpu as pltpu
from jax.experimental.pallas import tpu_sc as plsc

N_SEG = 1_000_000
U_DIM = 8
N_QUERIES = 1_048_576
ROW = 16
M = N_SEG // ROW
U2 = N_SEG * U_DIM // 16
NC, NS, L = 2, 16, 16
NW = NC * NS
QW = N_QUERIES // NW
B = 1024
NB = QW // B
CH = 128
NCH = B // CH

_COARSE_STEPS = 16
_LUT_STEPS = 7
_FINE_STEPS = 4


VPC = CH // L
ILP = 4
NBUCKET = 2048
BF = NBUCKET / 1000.0
NPV = (NBUCKET + L) // L


def _body(t_hbm, c_hbm, ts2_hbm, us16_hbm, out_hbm,
          c_v, tq, ridx, rows, gidx, par, urows, obuf, p_v,
          sem, semg, semt, semo):
    wid = lax.axis_index("s") * NC + lax.axis_index("c")
    pltpu.sync_copy(c_hbm, c_v)
    lanes = lax.iota(jnp.int32, L)
    qbase = wid * QW
    OB = B * U_DIM

    def t_copy(b, slot):
        return pltpu.make_async_copy(
            t_hbm.at[pl.ds(qbase + b * B, B)],
            tq.at[pl.ds(slot * B, B)], semt.at[slot])

    def o_copy(b, slot):
        return pltpu.make_async_copy(
            obuf.at[pl.ds(slot * OB, OB)],
            out_hbm.at[pl.ds((qbase + b * B) * U_DIM, OB)], semo.at[slot])

    def p_vec(v, _):
        qv = v * L + lanes
        lo = jnp.zeros((L,), jnp.int32)
        hi = jnp.full((L,), M, jnp.int32)
        for _s in range(_COARSE_STEPS):
            mid = jnp.minimum((lo + hi) >> 1, M - 1)
            bm = (plsc.load_gather(c_v, [mid]) * BF).astype(jnp.int32)
            le = bm < qv
            lo = jnp.where(le, mid + 1, lo)
            hi = jnp.where(le, hi, mid)
        p_v[pl.ds(v * L, L)] = lo
        return 0

    lax.fori_loop(0, NPV, p_vec, 0)

    t_copy(0, 0).start()

    def batch_body(b, _):
        tslot = b & 1
        qoff = qbase + b * B

        @pl.when(b + 1 < NB)
        def _():
            t_copy(b + 1, 1 - tslot).start()

        t_copy(b, tslot).wait()

        @pl.when(b >= 2)
        def _():
            o_copy(b - 2, tslot).wait()

        tq0 = tslot * B
        ob0 = tslot * OB

        def coarse_chunk(ch, _):
            for g in range(VPC // ILP):
                v0 = ch * VPC + g * ILP
                tv = [tq[pl.ds(tq0 + (v0 + i) * L, L)] for i in range(ILP)]
                qb = [(tv[i] * BF).astype(jnp.int32) for i in range(ILP)]
                lo0 = tuple(plsc.load_gather(p_v, [qb[i]]) for i in range(ILP))
                hi0 = tuple(plsc.load_gather(p_v, [qb[i] + 1]) for i in range(ILP))

                lo, hi = list(lo0), list(hi0)
                for _s in range(_LUT_STEPS):
                    for i in range(ILP):
                        mid = jnp.minimum((lo[i] + hi[i]) >> 1, M - 1)
                        le = plsc.load_gather(c_v, [mid]) <= tv[i]
                        lo[i] = jnp.where(le, mid + 1, lo[i])
                        hi[i] = jnp.where(le, hi[i], mid)

                def w_cond(carry):
                    lo, hi = carry
                    d = hi[0] - lo[0]
                    for i in range(1, ILP):
                        d = jnp.maximum(d, hi[i] - lo[i])
                    return jnp.max(d) > 0

                def w_body(carry):
                    lo, hi = carry
                    lo, hi = list(lo), list(hi)
                    for i in range(ILP):
                        mid = jnp.minimum((lo[i] + hi[i]) >> 1, M - 1)
                        le = plsc.load_gather(c_v, [mid]) <= tv[i]
                        lo[i] = jnp.where(le, mid + 1, lo[i])
                        hi[i] = jnp.where(le, hi[i], mid)
                    return tuple(lo), tuple(hi)

                lo, hi = lax.while_loop(w_cond, w_body,
                                        (tuple(lo), tuple(hi)))
                for i in range(ILP):
                    ridx[pl.ds((v0 + i) * L, L)] = jnp.maximum(lo[i] - 1, 0)
            pltpu.async_copy(ts2_hbm.at[ridx.at[pl.ds(ch * CH, CH)]],
                             rows.at[pl.ds(ch * CH, CH)], sem.at[ch])
            return 0

        lax.fori_loop(0, NCH, coarse_chunk, 0)

        def fine_chunk(ch, _):
            pltpu.make_async_copy(ts2_hbm.at[ridx.at[pl.ds(ch * CH, CH)]],
                                  rows.at[pl.ds(ch * CH, CH)],
                                  sem.at[ch]).wait()
            for g in range(VPC // ILP):
                v0 = ch * VPC + g * ILP
                tv = [tq[pl.ds(tq0 + (v0 + i) * L, L)] for i in range(ILP)]
                qv = [(v0 + i) * L + lanes for i in range(ILP)]
                lo = [jnp.ones((L,), jnp.int32)] * ILP
                hi = [jnp.full((L,), ROW, jnp.int32)] * ILP
                for _s in range(_FINE_STEPS):
                    for i in range(ILP):
                        mid = jnp.minimum((lo[i] + hi[i]) >> 1, ROW - 1)
                        le = plsc.load_gather(rows, [qv[i], mid]) <= tv[i]
                        lo[i] = jnp.where(le, mid + 1, lo[i])
                        hi[i] = jnp.where(le, hi[i], mid)
                for i in range(ILP):
                    rc = ridx[pl.ds((v0 + i) * L, L)]
                    fi = jnp.clip(rc * ROW + lo[i] - 1, 0, N_SEG - 1)
                    gidx[pl.ds((v0 + i) * L, L)] = fi >> 1
                    par[pl.ds((v0 + i) * L, L)] = (fi & 1) << 3
            pltpu.async_copy(us16_hbm.at[gidx.at[pl.ds(ch * CH, CH)]],
                             urows.at[pl.ds(ch * CH, CH)], semg.at[ch])
            return 0

        lax.fori_loop(0, NCH, fine_chunk, 0)

        def out_chunk(ch, _):
            pltpu.make_async_copy(us16_hbm.at[gidx.at[pl.ds(ch * CH, CH)]],
                                  urows.at[pl.ds(ch * CH, CH)],
                                  semg.at[ch]).wait()

            def vec_out(v, _):
                qv = v * L + lanes
                pv = par[pl.ds(v * L, L)]
                base = (v >> 3) * 1024 + (v & 7) * L
                for j in range(U_DIM):
                    val = plsc.load_gather(urows, [qv, pv + j])
                    obuf[pl.ds(ob0 + base + j * 128, L)] = val
                return 0

            lax.fori_loop(ch * VPC, (ch + 1) * VPC, vec_out, 0)
            return 0

        lax.fori_loop(0, NCH, out_chunk, 0)
        o_copy(b, tslot).start()
        return 0

    lax.fori_loop(0, NB, batch_body, 0)
    o_copy(NB - 2, (NB - 2) & 1).wait()
    o_copy(NB - 1, (NB - 1) & 1).wait()


NBLK = N_SEG // 128
BASE_BLK = NBLK // NW
EXTRA = NBLK - BASE_BLK * NW
TAIL_ROWS = N_SEG - NBLK * 128
TAIL_F = TAIL_ROWS * U_DIM
TAIL_OFF = NBLK * 128 * U_DIM


BPC = 16
FULL_CHUNKS = BASE_BLK // BPC
REM_BLK = BASE_BLK - FULL_CHUNKS * BPC


def _relayout_body(usT_hbm, tail_hbm, out_hbm, ub, ob, tb, semi, semo):
    wid = lax.axis_index("s") * NC + lax.axis_index("c")
    start = wid * BASE_BLK
    lanes = lax.iota(jnp.int32, L)
    jl = lanes & 7
    col0 = lanes >> 3

    chunks = [(start + i * BPC, BPC) for i in range(FULL_CHUNKS)]
    chunks.append((start + FULL_CHUNKS * BPC, REM_BLK))

    def in_copy(idx):
        k0, nb = chunks[idx]
        slot = idx & 1
        return pltpu.make_async_copy(
            usT_hbm.at[:, pl.ds(k0 * 128, nb * 128)],
            ub.at[slot, :, pl.ds(0, nb * 128)], semi.at[slot])

    def out_copy(idx):
        k0, nb = chunks[idx]
        slot = idx & 1
        return pltpu.make_async_copy(
            ob.at[slot, pl.ds(0, nb * 1024)],
            out_hbm.at[pl.ds(k0 * 1024, nb * 1024)], semo.at[slot])

    def transpose(slot, nb):
        def vec_body(o, _):
            for u in range(8):
                v = o * 8 + u
                kk = v >> 6
                ob[slot, pl.ds(v * L, L)] = plsc.load_gather(
                    ub.at[slot], [jl, col0 + 2 * (v & 63) + (kk << 7)])
            return 0

        lax.fori_loop(0, nb * 8, vec_body, 0)

    in_copy(0).start()
    for idx in range(len(chunks)):
        if idx + 1 < len(chunks):
            in_copy(idx + 1).start()
        in_copy(idx).wait()
        if idx >= 2:
            out_copy(idx - 2).wait()
        transpose(idx & 1, chunks[idx][1])
        out_copy(idx).start()
    out_copy(len(chunks) - 2).wait()
    out_copy(len(chunks) - 1).wait()

    @pl.when(wid >= NW - EXTRA)
    def _():
        k = NW * BASE_BLK + (wid - (NW - EXTRA))
        pltpu.sync_copy(usT_hbm.at[:, pl.ds(k * 128, 128)],
                        ub.at[0, :, pl.ds(0, 128)])
        transpose(0, 1)
        pltpu.sync_copy(ob.at[0, pl.ds(0, 1024)],
                        out_hbm.at[pl.ds(k * 1024, 1024)])

    @pl.when(wid == NW - 1)
    def _():
        pltpu.sync_copy(tail_hbm, tb)
        pltpu.sync_copy(tb, out_hbm.at[pl.ds(TAIL_OFF, TAIL_F)])


@jax.jit
def _relayout(usT, tail):
    mesh = plsc.VectorSubcoreMesh(core_axis_name="c", subcore_axis_name="s")
    return pl.kernel(
        _relayout_body,
        out_type=jax.ShapeDtypeStruct((N_SEG * U_DIM,), jnp.float32),
        mesh=mesh,
        compiler_params=pltpu.CompilerParams(
            needs_layout_passes=False, use_tc_tiling_on_sc=True),
        scratch_types=[
            pltpu.VMEM((2, U_DIM, BPC * 128 + 2), jnp.float32),
            pltpu.VMEM((2, BPC * 1024), jnp.float32),
            pltpu.VMEM((TAIL_F,), jnp.float32),
            pltpu.SemaphoreType.DMA((2,)),
            pltpu.SemaphoreType.DMA((2,)),
        ],
    )(usT, tail)


@jax.jit
def _run(t, c, ts2, us16):
    mesh = plsc.VectorSubcoreMesh(core_axis_name="c", subcore_axis_name="s")
    return pl.kernel(
        _body,
        out_type=jax.ShapeDtypeStruct((N_QUERIES * U_DIM,), jnp.float32),
        mesh=mesh,
        compiler_params=pltpu.CompilerParams(
            needs_layout_passes=False, use_tc_tiling_on_sc=False),
        scratch_types=[
            pltpu.VMEM((M,), jnp.float32),
            pltpu.VMEM((2 * B,), jnp.float32),
            pltpu.VMEM((B,), jnp.int32),
            pltpu.VMEM((B, ROW), jnp.float32),
            pltpu.VMEM((B,), jnp.int32),
            pltpu.VMEM((B,), jnp.int32),
            pltpu.VMEM((B, 16), jnp.float32),
            pltpu.VMEM((2 * B * U_DIM,), jnp.float32),
            pltpu.VMEM((NPV * L,), jnp.int32),
            pltpu.SemaphoreType.DMA((NCH,)),
            pltpu.SemaphoreType.DMA((NCH,)),
            pltpu.SemaphoreType.DMA((2,)),
            pltpu.SemaphoreType.DMA((2,)),
        ],
    )(t, c, ts2, us16)


def kernel(t, x, ts, us):
    c = ts[::ROW]
    ts2 = ts.reshape(M, ROW)
    usT = us.T
    tail = us[NBLK * 128:].reshape(TAIL_F)
    us16 = _relayout(usT, tail).reshape(U2, 16)
    o = _run(t, c, ts2, us16)
    o3 = o.reshape(N_QUERIES // 128, U_DIM, 128)
    return o3.transpose(0, 2, 1).reshape(N_QUERIES, U_DIM)
